# Initial kernel scaffold; baseline (speedup 1.0000x reference)
import jax
import jax.numpy as jnp
from jax.experimental import pallas as pl


def kernel(x, edge_index, W1, b1, W2, b2):
    raise NotImplementedError("write your pallas kernel here")


# speedup vs baseline: 9.8453x; 9.8453x over previous
import functools

import jax
import jax.numpy as jnp
from jax import lax
from jax.experimental import pallas as pl
from jax.experimental.pallas import tpu as pltpu
from jax.experimental.pallas import tpu_sc as plsc

N = 10000
IN_CH = 128
HID = 256
OUT = 10
E = 320000

CHUNK = 128
NSUB = 16
NCORE = 2
NW = NCORE * NSUB
EPAD = 323584
NPAD = 10112
DUMP = N
BM = 1000
NB = N // BM
CB_ALL = EPAD // (NSUB * CHUNK)
CB_SPLIT = EPAD // (NW * CHUNK)
RSUB_PAD = NPAD // NSUB
RSUB_OUT = 632
RSUB_TAIL = N - 15 * RSUB_OUT


def _copy_out(acc_sh, out_hbm, c, s):
    @pl.when(s < NSUB - 1)
    def _():
        o0 = s * RSUB_OUT
        pltpu.sync_copy(acc_sh.at[pl.ds(o0, RSUB_OUT)],
                        out_hbm.at[pl.ds(c * N + o0, RSUB_OUT)])

    @pl.when(s == NSUB - 1)
    def _():
        o0 = (NSUB - 1) * RSUB_OUT
        pltpu.sync_copy(acc_sh.at[pl.ds(o0, RSUB_TAIL)],
                        out_hbm.at[pl.ds(c * N + o0, RSUB_TAIL)])

@functools.cache
def _vmesh():
    return plsc.VectorSubcoreMesh(core_axis_name="c", subcore_axis_name="s",
                                  num_cores=NCORE, num_subcores=NSUB)


def _sc_deg(dst_pad, ones16, z16):
    @functools.partial(
        pl.kernel,
        out_type=jax.ShapeDtypeStruct((2 * N, 16), jnp.float32),
        mesh=_vmesh(),
        scratch_types=[
            pltpu.VMEM((1, CHUNK), jnp.int32),
            pltpu.VMEM((CHUNK, 16), jnp.float32),
            pltpu.VMEM_SHARED((NPAD, 16), jnp.float32),
            pltpu.SemaphoreType.DMA,
        ],
    )
    def k(dst_hbm, ones_hbm, z_hbm, out_hbm, didx_v, ones_v, acc_sh, sem):
        c = lax.axis_index("c")
        s = lax.axis_index("s")
        r0 = s * RSUB_PAD
        pltpu.sync_copy(z_hbm.at[pl.ds(r0, RSUB_PAD)],
                        acc_sh.at[pl.ds(r0, RSUB_PAD)])
        pltpu.sync_copy(ones_hbm, ones_v)
        plsc.subcore_barrier()
        base = (c * NSUB + s) * CB_SPLIT * CHUNK

        @pl.loop(0, CB_SPLIT)
        def _(i):
            off = base + i * CHUNK
            pltpu.sync_copy(dst_hbm.at[pl.ds(off, CHUNK)], didx_v.at[0])
            pltpu.sync_copy(ones_v, acc_sh.at[didx_v.at[0]], add=True)

        plsc.subcore_barrier()
        _copy_out(acc_sh, out_hbm, c, s)

    return k(dst_pad, ones16, z16)


def _sc_agg(y_flat, src_flat, dst_pad, z128, all_edges):
    cb = CB_ALL if all_edges else CB_SPLIT

    @functools.partial(
        pl.kernel,
        out_type=jax.ShapeDtypeStruct((2 * N, IN_CH), jnp.float32),
        mesh=_vmesh(),
        scratch_types=[
            pltpu.VMEM((CHUNK,), jnp.int32),
            pltpu.VMEM((1, CHUNK), jnp.int32),
            pltpu.VMEM((CHUNK, IN_CH), jnp.float32),
            pltpu.VMEM_SHARED((NPAD, IN_CH), jnp.float32),
            pltpu.SemaphoreType.DMA,
        ],
    )
    def k(y_hbm, src_hbm, dst_hbm, z_hbm, out_hbm,
          sidx_v, didx_v, rows_v, acc_sh, sem):
        c = lax.axis_index("c")
        s = lax.axis_index("s")
        r0 = s * RSUB_PAD
        pltpu.sync_copy(z_hbm.at[pl.ds(r0, RSUB_PAD)],
                        acc_sh.at[pl.ds(r0, RSUB_PAD)])
        plsc.subcore_barrier()
        if all_edges:
            ebase = s * cb * CHUNK
            sbase = c * EPAD + ebase
        else:
            ebase = (c * NSUB + s) * cb * CHUNK
            sbase = ebase

        @pl.loop(0, cb)
        def _(i):
            pltpu.sync_copy(src_hbm.at[pl.ds(sbase + i * CHUNK, CHUNK)], sidx_v)
            pltpu.sync_copy(dst_hbm.at[pl.ds(ebase + i * CHUNK, CHUNK)],
                            didx_v.at[0])
            pltpu.async_copy(y_hbm.at[sidx_v], rows_v, sem).wait()
            pltpu.sync_copy(rows_v, acc_sh.at[didx_v.at[0]], add=True)

        plsc.subcore_barrier()
        _copy_out(acc_sh, out_hbm, c, s)

    return k(y_flat, src_flat, dst_pad, z128)


def _tc_matmul1(x, W1):
    def body(x_ref, w_ref, o_ref):
        o_ref[...] = jnp.dot(x_ref[...], w_ref[...],
                             preferred_element_type=jnp.float32)

    return pl.pallas_call(
        body,
        grid=(2 * NB,),
        in_specs=[
            pl.BlockSpec((BM, IN_CH), lambda i: (i % NB, 0)),
            pl.BlockSpec((IN_CH, IN_CH), lambda i: (0, i // NB)),
        ],
        out_specs=pl.BlockSpec((BM, IN_CH), lambda i: (i, 0)),
        out_shape=jax.ShapeDtypeStruct((2 * N, IN_CH), jnp.float32),
    )(x, W1)


def _dinv_block(dlo, dhi):
    deg = dlo[...][:, :1] + dhi[...][:, :1] + 1.0
    return lax.rsqrt(deg)


def _tc_scale(deg_part, xw_flat):
    def body(dlo, dhi, xw_ref, o_ref):
        o_ref[...] = xw_ref[...] * _dinv_block(dlo, dhi)

    return pl.pallas_call(
        body,
        grid=(2 * NB,),
        in_specs=[
            pl.BlockSpec((BM, 16), lambda i: (i % NB, 0)),
            pl.BlockSpec((BM, 16), lambda i: (i % NB + NB, 0)),
            pl.BlockSpec((BM, IN_CH), lambda i: (i, 0)),
        ],
        out_specs=pl.BlockSpec((BM, IN_CH), lambda i: (i, 0)),
        out_shape=jax.ShapeDtypeStruct((2 * N, IN_CH), jnp.float32),
    )(deg_part, deg_part, xw_flat)


def _tc_layer2(deg_part, agg1_flat, y1_flat, b1r, W2pad):
    def body(dlo, dhi, alo, ahi, ylo, yhi, b_ref, w_ref, o_ref):
        dinv = _dinv_block(dlo, dhi)
        b = b_ref[...]
        h_lo = jnp.maximum(dinv * (alo[...] + ylo[...]) + b[:1, :IN_CH], 0.0)
        h_hi = jnp.maximum(dinv * (ahi[...] + yhi[...]) + b[:1, IN_CH:], 0.0)
        w = w_ref[...]
        xw2 = (jnp.dot(h_lo, w[:IN_CH], preferred_element_type=jnp.float32)
               + jnp.dot(h_hi, w[IN_CH:], preferred_element_type=jnp.float32))
        o_ref[...] = dinv * xw2

    return pl.pallas_call(
        body,
        grid=(NB,),
        in_specs=[
            pl.BlockSpec((BM, 16), lambda i: (i, 0)),
            pl.BlockSpec((BM, 16), lambda i: (i + NB, 0)),
            pl.BlockSpec((BM, IN_CH), lambda i: (i, 0)),
            pl.BlockSpec((BM, IN_CH), lambda i: (i + NB, 0)),
            pl.BlockSpec((BM, IN_CH), lambda i: (i, 0)),
            pl.BlockSpec((BM, IN_CH), lambda i: (i + NB, 0)),
            pl.BlockSpec((8, HID), lambda i: (0, 0)),
            pl.BlockSpec((HID, IN_CH), lambda i: (0, 0)),
        ],
        out_specs=pl.BlockSpec((BM, IN_CH), lambda i: (i, 0)),
        out_shape=jax.ShapeDtypeStruct((N, IN_CH), jnp.float32),
    )(deg_part, deg_part, agg1_flat, agg1_flat, y1_flat, y1_flat, b1r, W2pad)


def _tc_final(deg_part, agg2_flat, y2, b2r):
    def body(dlo, dhi, alo, ahi, y_ref, b_ref, o_ref):
        dinv = _dinv_block(dlo, dhi)
        o = dinv * (alo[...] + ahi[...] + y_ref[...]) + b_ref[:1, :]
        col = lax.broadcasted_iota(jnp.int32, o.shape, 1)
        om = jnp.where(col < OUT, o, -1e30)
        m = jnp.max(om, axis=1, keepdims=True)
        lse = jnp.log(jnp.sum(jnp.exp(om - m), axis=1, keepdims=True))
        o_ref[...] = o - m - lse

    return pl.pallas_call(
        body,
        grid=(NB,),
        in_specs=[
            pl.BlockSpec((BM, 16), lambda i: (i, 0)),
            pl.BlockSpec((BM, 16), lambda i: (i + NB, 0)),
            pl.BlockSpec((BM, IN_CH), lambda i: (i, 0)),
            pl.BlockSpec((BM, IN_CH), lambda i: (i + NB, 0)),
            pl.BlockSpec((BM, IN_CH), lambda i: (i, 0)),
            pl.BlockSpec((8, IN_CH), lambda i: (0, 0)),
        ],
        out_specs=pl.BlockSpec((BM, IN_CH), lambda i: (i, 0)),
        out_shape=jax.ShapeDtypeStruct((N, IN_CH), jnp.float32),
    )(deg_part, deg_part, agg2_flat, agg2_flat, y2, b2r)


def kernel(x, edge_index, W1, b1, W2, b2):
    ei = edge_index.astype(jnp.int32)
    src = ei[0]
    dst = ei[1]
    pad = EPAD - E
    src_pad = jnp.concatenate([src, jnp.zeros((pad,), jnp.int32)])
    dst_pad = jnp.concatenate([dst, jnp.full((pad,), DUMP, jnp.int32)])
    src2 = jnp.concatenate([src_pad, src_pad + N])

    z16 = jnp.zeros((NPAD, 16), jnp.float32)
    z128 = jnp.zeros((NPAD, IN_CH), jnp.float32)
    ones16 = jnp.ones((CHUNK, 16), jnp.float32)
    b1r = jnp.broadcast_to(b1.reshape(1, HID), (8, HID))
    W2pad = jnp.zeros((HID, IN_CH), W2.dtype).at[:, :OUT].set(W2)
    b2r = jnp.broadcast_to(
        jnp.zeros((1, IN_CH), b2.dtype).at[0, :OUT].set(b2), (8, IN_CH))

    deg_part = _sc_deg(dst_pad, ones16, z16)
    xw_flat = _tc_matmul1(x, W1)
    y1_flat = _tc_scale(deg_part, xw_flat)
    agg1_flat = _sc_agg(y1_flat, src2, dst_pad, z128, all_edges=True)
    y2 = _tc_layer2(deg_part, agg1_flat, y1_flat, b1r, W2pad)
    agg2_flat = _sc_agg(y2, src_pad, dst_pad, z128, all_edges=False)
    out128 = _tc_final(deg_part, agg2_flat, y2, b2r)
    return out128[:, :OUT]
